# baseline (speedup 1.0000x reference)
import jax
import jax.numpy as jnp
from jax.experimental import pallas as pl

NB, NT, NF, NAZI = 32, 256, 256, 180
ROWS = NB * NT
TILE = 2048
TB = TILE // NT
RAD2DEG = 180.0 / 3.141592653589793


def _fused_kernel(xre_ref, xim_ref, wre_ref, wim_ref, azi_ref,
                  azigt_ref, vad_ref, ss_ref, doa_ref, acc_ref):
    i = pl.program_id(0)
    ss = (jnp.dot(xre_ref[...], wre_ref[...], preferred_element_type=jnp.float32)
          + jnp.dot(xim_ref[...], wim_ref[...], preferred_element_type=jnp.float32))
    ss_ref[...] = ss

    azi = azi_ref[...]
    mx = jnp.max(ss, axis=1, keepdims=True)
    doa = jnp.min(jnp.where(ss >= mx, azi, jnp.inf), axis=1, keepdims=True)
    doa2 = doa.reshape(TB, NT)
    doa_ref[...] = doa2

    azi_gt = azigt_ref[...]
    vad = (vad_ref[...] > 0.5).astype(jnp.float32)
    err = jnp.abs(doa2 - azi_gt) * RAD2DEG
    err = jnp.minimum(err, 360.0 - err)
    corr = (err < 30.0).astype(jnp.float32) * vad
    lane = jax.lax.broadcasted_iota(jnp.int32, (1, 128), 1)
    part = (jnp.where(lane == 0, jnp.sum(corr), 0.0)
            + jnp.where(lane == 1, jnp.sum(vad * err), 0.0)
            + jnp.where(lane == 2, jnp.sum(vad), 0.0))

    @pl.when(i == 0)
    def _init():
        acc_ref[...] = part

    @pl.when(i > 0)
    def _accum():
        acc_ref[...] += part


def kernel(pred_batch, doa_gt, vad_gt, tmpl_re, tmpl_im, azi_grid):
    x_re = pred_batch[:, :, 0, :].reshape(ROWS, NF)
    x_im = pred_batch[:, :, 1, :].reshape(ROWS, NF)
    w_re = tmpl_re.T
    w_im = tmpl_im.T
    azi2 = azi_grid.reshape(1, NAZI)
    azi_gt = doa_gt[:, :, 1, 0]
    vad2 = vad_gt[:, :, 0]

    grid = (ROWS // TILE,)
    ss, doa, acc = pl.pallas_call(
        _fused_kernel,
        grid=grid,
        in_specs=[
            pl.BlockSpec((TILE, NF), lambda i: (i, 0)),
            pl.BlockSpec((TILE, NF), lambda i: (i, 0)),
            pl.BlockSpec((NF, NAZI), lambda i: (0, 0)),
            pl.BlockSpec((NF, NAZI), lambda i: (0, 0)),
            pl.BlockSpec((1, NAZI), lambda i: (0, 0)),
            pl.BlockSpec((TB, NT), lambda i: (i, 0)),
            pl.BlockSpec((TB, NT), lambda i: (i, 0)),
        ],
        out_specs=[
            pl.BlockSpec((TILE, NAZI), lambda i: (i, 0)),
            pl.BlockSpec((TB, NT), lambda i: (i, 0)),
            pl.BlockSpec((1, 128), lambda i: (0, 0)),
        ],
        out_shape=[
            jax.ShapeDtypeStruct((ROWS, NAZI), jnp.float32),
            jax.ShapeDtypeStruct((NB, NT), jnp.float32),
            jax.ShapeDtypeStruct((1, 128), jnp.float32),
        ],
    )(x_re, x_im, w_re, w_im, azi2, azi_gt, vad2)

    ss = ss.reshape(NB, NT, NAZI)
    denom = jnp.maximum(acc[0, 2], 1.0)
    ACC = acc[0, 0] / denom
    MAE = acc[0, 1] / denom
    return ss, doa, ACC, MAE


# speedup vs baseline: 1.2734x; 1.2734x over previous
import jax
import jax.numpy as jnp
from jax.experimental import pallas as pl

NB, NT, NF, NAZI = 32, 256, 256, 180
ROWS = NB * NT
TILE = 2048
TB = TILE // NT
RAD2DEG = 180.0 / 3.141592653589793


def _fused_kernel(xre_ref, xim_ref, wre_ref, wim_ref, azi_ref,
                  azigt_ref, vad_ref, sst_ref, doa_ref, acc_ref):
    i = pl.program_id(0)
    x_re = xre_ref[...].reshape(TILE, NF)
    x_im = xim_ref[...].reshape(TILE, NF)
    dims = (((1,), (1,)), ((), ()))
    sst = (jax.lax.dot_general(wre_ref[...], x_re, dims,
                               preferred_element_type=jnp.float32)
           + jax.lax.dot_general(wim_ref[...], x_im, dims,
                                 preferred_element_type=jnp.float32))
    sst_ref[...] = sst

    azi = azi_ref[...]
    mx = jnp.max(sst, axis=0, keepdims=True)
    doa = jnp.min(jnp.where(sst >= mx, azi, jnp.inf), axis=0, keepdims=True)
    doa2 = doa.reshape(TB, NT)
    doa_ref[...] = doa2

    azi_gt = azigt_ref[...]
    vad = (vad_ref[...] > 0.5).astype(jnp.float32)
    err = jnp.abs(doa2 - azi_gt) * RAD2DEG
    err = jnp.minimum(err, 360.0 - err)
    corr = (err < 30.0).astype(jnp.float32) * vad
    lane = jax.lax.broadcasted_iota(jnp.int32, (1, 128), 1)
    part = (jnp.where(lane == 0, jnp.sum(corr), 0.0)
            + jnp.where(lane == 1, jnp.sum(vad * err), 0.0)
            + jnp.where(lane == 2, jnp.sum(vad), 0.0))

    @pl.when(i == 0)
    def _init():
        acc_ref[...] = part

    @pl.when(i > 0)
    def _accum():
        acc_ref[...] += part


def kernel(pred_batch, doa_gt, vad_gt, tmpl_re, tmpl_im, azi_grid):
    y = pred_batch.transpose(2, 0, 1, 3)
    azi_col = azi_grid.reshape(NAZI, 1)
    azi_gt = doa_gt[:, :, 1, 0]
    vad2 = vad_gt[:, :, 0]

    grid = (ROWS // TILE,)
    sst, doa, acc = pl.pallas_call(
        _fused_kernel,
        grid=grid,
        in_specs=[
            pl.BlockSpec((1, TB, NT, NF), lambda i: (0, i, 0, 0)),
            pl.BlockSpec((1, TB, NT, NF), lambda i: (1, i, 0, 0)),
            pl.BlockSpec((NAZI, NF), lambda i: (0, 0)),
            pl.BlockSpec((NAZI, NF), lambda i: (0, 0)),
            pl.BlockSpec((NAZI, 1), lambda i: (0, 0)),
            pl.BlockSpec((TB, NT), lambda i: (i, 0)),
            pl.BlockSpec((TB, NT), lambda i: (i, 0)),
        ],
        out_specs=[
            pl.BlockSpec((NAZI, TILE), lambda i: (0, i)),
            pl.BlockSpec((TB, NT), lambda i: (i, 0)),
            pl.BlockSpec((1, 128), lambda i: (0, 0)),
        ],
        out_shape=[
            jax.ShapeDtypeStruct((NAZI, ROWS), jnp.float32),
            jax.ShapeDtypeStruct((NB, NT), jnp.float32),
            jax.ShapeDtypeStruct((1, 128), jnp.float32),
        ],
    )(y, y, tmpl_re, tmpl_im, azi_col, azi_gt, vad2)

    ss = sst.reshape(NAZI, NB, NT).transpose(1, 2, 0)
    denom = jnp.maximum(acc[0, 2], 1.0)
    ACC = acc[0, 0] / denom
    MAE = acc[0, 1] / denom
    return ss, doa, ACC, MAE


# speedup vs baseline: 1.9154x; 1.5042x over previous
import jax
import jax.numpy as jnp
from jax.experimental import pallas as pl

NB, NT, NF, NAZI = 32, 256, 256, 180
RAD2DEG = 180.0 / 3.141592653589793


def _fused_kernel(x_ref, wre_ref, wim_ref, azi_ref, azigt_ref, vad_ref,
                  sst_ref, doa_ref, acc_ref):
    b = pl.program_id(0)
    b8 = jax.lax.rem(b, 8)
    x_re = x_ref[0, :, 0, :]
    x_im = x_ref[0, :, 1, :]
    dims = (((1,), (1,)), ((), ()))
    sst = (jax.lax.dot_general(wre_ref[...], x_re, dims,
                               preferred_element_type=jnp.float32)
           + jax.lax.dot_general(wim_ref[...], x_im, dims,
                                 preferred_element_type=jnp.float32))
    sst_ref[:, 0, pl.ds(b8, 1), :] = sst.reshape(NAZI, 1, NT)

    azi = azi_ref[...]
    mx = jnp.max(sst, axis=0, keepdims=True)
    doa = jnp.min(jnp.where(sst >= mx, azi, jnp.inf), axis=0, keepdims=True)
    doa_ref[pl.ds(b, 1), :] = doa

    azi_gt = azigt_ref[pl.ds(b, 1), :]
    vad = (vad_ref[pl.ds(b, 1), :] > 0.5).astype(jnp.float32)
    err = jnp.abs(doa - azi_gt) * RAD2DEG
    err = jnp.minimum(err, 360.0 - err)
    corr = (err < 30.0).astype(jnp.float32) * vad
    lane = jax.lax.broadcasted_iota(jnp.int32, (1, 128), 1)
    part = (jnp.where(lane == 0, jnp.sum(corr), 0.0)
            + jnp.where(lane == 1, jnp.sum(vad * err), 0.0)
            + jnp.where(lane == 2, jnp.sum(vad), 0.0))

    @pl.when(b == 0)
    def _init():
        acc_ref[...] = part

    @pl.when(b > 0)
    def _accum():
        acc_ref[...] += part


def kernel(pred_batch, doa_gt, vad_gt, tmpl_re, tmpl_im, azi_grid):
    azi_col = azi_grid.reshape(NAZI, 1)
    azi_gt = doa_gt[:, :, 1, 0]
    vad2 = vad_gt[:, :, 0]

    grid = (NB,)
    sst4, doa, acc = pl.pallas_call(
        _fused_kernel,
        grid=grid,
        in_specs=[
            pl.BlockSpec((1, NT, 2, NF), lambda b: (b, 0, 0, 0)),
            pl.BlockSpec((NAZI, NF), lambda b: (0, 0)),
            pl.BlockSpec((NAZI, NF), lambda b: (0, 0)),
            pl.BlockSpec((NAZI, 1), lambda b: (0, 0)),
            pl.BlockSpec((NB, NT), lambda b: (0, 0)),
            pl.BlockSpec((NB, NT), lambda b: (0, 0)),
        ],
        out_specs=[
            pl.BlockSpec((NAZI, 1, 8, NT), lambda b: (0, b // 8, 0, 0)),
            pl.BlockSpec((NB, NT), lambda b: (0, 0)),
            pl.BlockSpec((1, 128), lambda b: (0, 0)),
        ],
        out_shape=[
            jax.ShapeDtypeStruct((NAZI, NB // 8, 8, NT), jnp.float32),
            jax.ShapeDtypeStruct((NB, NT), jnp.float32),
            jax.ShapeDtypeStruct((1, 128), jnp.float32),
        ],
    )(pred_batch, tmpl_re, tmpl_im, azi_col, azi_gt, vad2)

    ss = sst4.reshape(NAZI, NB, NT).transpose(1, 2, 0)
    denom = jnp.maximum(acc[0, 2], 1.0)
    ACC = acc[0, 0] / denom
    MAE = acc[0, 1] / denom
    return ss, doa, ACC, MAE


# speedup vs baseline: 3.0373x; 1.5857x over previous
import jax
import jax.numpy as jnp
from jax.experimental import pallas as pl

NB, NT, NF, NAZI = 32, 256, 256, 180
GB = 8
RAD2DEG = 180.0 / 3.141592653589793


def _fused_kernel(x_ref, wre_ref, wim_ref, azi_ref, azigt_ref, vad_ref,
                  sst_ref, doa_ref, acc_ref):
    i = pl.program_id(0)
    w_re = wre_ref[...]
    w_im = wim_ref[...]
    azi = azi_ref[...]
    dims = (((1,), (1,)), ((), ()))
    doa_rows = []
    for bb in range(GB):
        x_re = x_ref[bb, :, 0, :]
        x_im = x_ref[bb, :, 1, :]
        sst = (jax.lax.dot_general(w_re, x_re, dims,
                                   preferred_element_type=jnp.float32)
               + jax.lax.dot_general(w_im, x_im, dims,
                                     preferred_element_type=jnp.float32))
        sst_ref[:, 0, bb, :] = sst
        mx = jnp.max(sst, axis=0, keepdims=True)
        doa_rows.append(jnp.min(jnp.where(sst >= mx, azi, jnp.inf),
                                axis=0, keepdims=True))
    doa = jnp.concatenate(doa_rows, axis=0)
    doa_ref[...] = doa

    azi_gt = azigt_ref[...]
    vad = (vad_ref[...] > 0.5).astype(jnp.float32)
    err = jnp.abs(doa - azi_gt) * RAD2DEG
    err = jnp.minimum(err, 360.0 - err)
    corr = (err < 30.0).astype(jnp.float32) * vad
    lane = jax.lax.broadcasted_iota(jnp.int32, (1, 128), 1)
    part = (jnp.where(lane == 0, jnp.sum(corr), 0.0)
            + jnp.where(lane == 1, jnp.sum(vad * err), 0.0)
            + jnp.where(lane == 2, jnp.sum(vad), 0.0))

    @pl.when(i == 0)
    def _init():
        acc_ref[...] = part

    @pl.when(i > 0)
    def _accum():
        acc_ref[...] += part


def kernel(pred_batch, doa_gt, vad_gt, tmpl_re, tmpl_im, azi_grid):
    azi_col = azi_grid.reshape(NAZI, 1)
    azi_gt = doa_gt[:, :, 1, 0]
    vad2 = vad_gt[:, :, 0]

    grid = (NB // GB,)
    sst4, doa, acc = pl.pallas_call(
        _fused_kernel,
        grid=grid,
        in_specs=[
            pl.BlockSpec((GB, NT, 2, NF), lambda i: (i, 0, 0, 0)),
            pl.BlockSpec((NAZI, NF), lambda i: (0, 0)),
            pl.BlockSpec((NAZI, NF), lambda i: (0, 0)),
            pl.BlockSpec((NAZI, 1), lambda i: (0, 0)),
            pl.BlockSpec((GB, NT), lambda i: (i, 0)),
            pl.BlockSpec((GB, NT), lambda i: (i, 0)),
        ],
        out_specs=[
            pl.BlockSpec((NAZI, 1, GB, NT), lambda i: (0, i, 0, 0)),
            pl.BlockSpec((GB, NT), lambda i: (i, 0)),
            pl.BlockSpec((1, 128), lambda i: (0, 0)),
        ],
        out_shape=[
            jax.ShapeDtypeStruct((NAZI, NB // GB, GB, NT), jnp.float32),
            jax.ShapeDtypeStruct((NB, NT), jnp.float32),
            jax.ShapeDtypeStruct((1, 128), jnp.float32),
        ],
    )(pred_batch, tmpl_re, tmpl_im, azi_col, azi_gt, vad2)

    ss = sst4.reshape(NAZI, NB, NT).transpose(1, 2, 0)
    denom = jnp.maximum(acc[0, 2], 1.0)
    ACC = acc[0, 0] / denom
    MAE = acc[0, 1] / denom
    return ss, doa, ACC, MAE


# speedup vs baseline: 4.4305x; 1.4587x over previous
import math

import jax
import jax.numpy as jnp
from jax.experimental import pallas as pl
from jax.experimental.pallas import tpu as pltpu

NB, NT, NF, NAZI = 32, 256, 256, 180
GB = 8
TH = NT // 2
RAD2DEG = 180.0 / math.pi
AZI_STEP = math.pi / (NAZI - 1)


def _fused_kernel(x_ref, wre_ref, wim_ref, gt_ref, vad_ref,
                  sst_ref, doa_ref, acc_a_ref, mae_a_ref, acc_ref):
    i = pl.program_id(0)
    j = pl.program_id(1)
    w_re = wre_ref[...]
    w_im = wim_ref[...]
    azi = (jax.lax.broadcasted_iota(jnp.int32, (NAZI, 1), 0)
           .astype(jnp.float32) * AZI_STEP)
    dims = (((1,), (1,)), ((), ()))
    doa_rows = []
    for bb in range(GB):
        x_re = x_ref[bb, :, 0, :]
        x_im = x_ref[bb, :, 1, :]
        sst = (jax.lax.dot_general(w_re, x_re, dims,
                                   preferred_element_type=jnp.float32)
               + jax.lax.dot_general(w_im, x_im, dims,
                                     preferred_element_type=jnp.float32))
        sst_ref[:, 0, bb, :] = sst
        mx = jnp.max(sst, axis=0, keepdims=True)
        doa_rows.append(jnp.min(jnp.where(sst >= mx, azi, jnp.inf),
                                axis=0, keepdims=True))
    doa = jnp.concatenate(doa_rows, axis=0)
    doa_ref[...] = doa

    azi_gt = gt_ref[:, 1, 0, :]
    vad = (vad_ref[:, 0, :] > 0.5).astype(jnp.float32)
    err = jnp.abs(doa - azi_gt) * RAD2DEG
    err = jnp.minimum(err, 360.0 - err)
    corr = (err < 30.0).astype(jnp.float32) * vad
    lane = jax.lax.broadcasted_iota(jnp.int32, (1, 128), 1)
    part = (jnp.where(lane == 0, jnp.sum(corr), 0.0)
            + jnp.where(lane == 1, jnp.sum(vad * err), 0.0)
            + jnp.where(lane == 2, jnp.sum(vad), 0.0))

    first = jnp.logical_and(i == 0, j == 0)
    last = jnp.logical_and(i == NB // GB - 1, j == NT // TH - 1)

    @pl.when(first)
    def _init():
        acc_ref[...] = part

    @pl.when(jnp.logical_not(first))
    def _accum():
        acc_ref[...] += part

    @pl.when(last)
    def _final():
        tot = acc_ref[...]
        denom = jnp.maximum(tot[0, 2], 1.0)
        acc_a_ref[...] = tot[0:1, 0:1] / denom
        mae_a_ref[...] = tot[0:1, 1:2] / denom


def kernel(pred_batch, doa_gt, vad_gt, tmpl_re, tmpl_im, azi_grid):
    del azi_grid
    gt3 = doa_gt.transpose(0, 2, 3, 1)
    vad3 = vad_gt.transpose(0, 2, 1)

    grid = (NB // GB, NT // TH)
    sst4, doa, acc_a, mae_a = pl.pallas_call(
        _fused_kernel,
        grid=grid,
        in_specs=[
            pl.BlockSpec((GB, TH, 2, NF), lambda i, j: (i, j, 0, 0)),
            pl.BlockSpec((NAZI, NF), lambda i, j: (0, 0)),
            pl.BlockSpec((NAZI, NF), lambda i, j: (0, 0)),
            pl.BlockSpec((GB, 2, 1, TH), lambda i, j: (i, 0, 0, j)),
            pl.BlockSpec((GB, 1, TH), lambda i, j: (i, 0, j)),
        ],
        out_specs=[
            pl.BlockSpec((NAZI, 1, GB, TH), lambda i, j: (0, i, 0, j)),
            pl.BlockSpec((GB, TH), lambda i, j: (i, j)),
            pl.BlockSpec((1, 1), lambda i, j: (0, 0)),
            pl.BlockSpec((1, 1), lambda i, j: (0, 0)),
        ],
        out_shape=[
            jax.ShapeDtypeStruct((NAZI, NB // GB, GB, NT), jnp.float32),
            jax.ShapeDtypeStruct((NB, NT), jnp.float32),
            jax.ShapeDtypeStruct((1, 1), jnp.float32),
            jax.ShapeDtypeStruct((1, 1), jnp.float32),
        ],
        scratch_shapes=[pltpu.VMEM((1, 128), jnp.float32)],
    )(pred_batch, tmpl_re, tmpl_im, gt3, vad3)

    ss = sst4.reshape(NAZI, NB, NT).transpose(1, 2, 0)
    return ss, doa, acc_a.reshape(()), mae_a.reshape(())


# speedup vs baseline: 4.5684x; 1.0311x over previous
import math

import jax
import jax.numpy as jnp
from jax.experimental import pallas as pl
from jax.experimental.pallas import tpu as pltpu

NB, NT, NF, NAZI = 32, 256, 256, 180
GB = 8
TH = NT // 2
RAD2DEG = 180.0 / math.pi
AZI_STEP = math.pi / (NAZI - 1)


def _fused_kernel(x_ref, wre_ref, wim_ref, gt_ref, vad_ref,
                  sst_ref, doa_ref, acc_a_ref, mae_a_ref, acc_ref):
    i = pl.program_id(0)
    j = pl.program_id(1)
    w_re = wre_ref[...]
    w_im = wim_ref[...]
    azi = (jax.lax.broadcasted_iota(jnp.int32, (NAZI, 1), 0)
           .astype(jnp.float32) * AZI_STEP)
    dims = (((1,), (1,)), ((), ()))
    x_re = x_ref[:, :, 0, :].reshape(GB * TH, NF)
    x_im = x_ref[:, :, 1, :].reshape(GB * TH, NF)
    sst_all = (jax.lax.dot_general(w_re, x_re, dims,
                                   preferred_element_type=jnp.float32)
               + jax.lax.dot_general(w_im, x_im, dims,
                                     preferred_element_type=jnp.float32))
    mx = jnp.max(sst_all, axis=0, keepdims=True)
    doa_all = jnp.min(jnp.where(sst_all >= mx, azi, jnp.inf),
                      axis=0, keepdims=True)
    doa_rows = []
    for bb in range(GB):
        sst_ref[:, 0, bb, :] = sst_all[:, bb * TH:(bb + 1) * TH]
        doa_rows.append(doa_all[:, bb * TH:(bb + 1) * TH])
    doa = jnp.concatenate(doa_rows, axis=0)
    doa_ref[...] = doa

    azi_gt = gt_ref[:, 1, 0, :]
    vad = (vad_ref[:, 0, :] > 0.5).astype(jnp.float32)
    err = jnp.abs(doa - azi_gt) * RAD2DEG
    err = jnp.minimum(err, 360.0 - err)
    corr = (err < 30.0).astype(jnp.float32) * vad
    lane = jax.lax.broadcasted_iota(jnp.int32, (1, 128), 1)
    part = (jnp.where(lane == 0, jnp.sum(corr), 0.0)
            + jnp.where(lane == 1, jnp.sum(vad * err), 0.0)
            + jnp.where(lane == 2, jnp.sum(vad), 0.0))

    first = jnp.logical_and(i == 0, j == 0)
    last = jnp.logical_and(i == NB // GB - 1, j == NT // TH - 1)

    @pl.when(first)
    def _init():
        acc_ref[...] = part

    @pl.when(jnp.logical_not(first))
    def _accum():
        acc_ref[...] += part

    @pl.when(last)
    def _final():
        tot = acc_ref[...]
        denom = jnp.maximum(tot[0, 2], 1.0)
        acc_a_ref[...] = tot[0:1, 0:1] / denom
        mae_a_ref[...] = tot[0:1, 1:2] / denom


def kernel(pred_batch, doa_gt, vad_gt, tmpl_re, tmpl_im, azi_grid):
    del azi_grid
    gt3 = doa_gt.transpose(0, 2, 3, 1)
    vad3 = vad_gt.transpose(0, 2, 1)

    grid = (NB // GB, NT // TH)
    sst4, doa, acc_a, mae_a = pl.pallas_call(
        _fused_kernel,
        grid=grid,
        in_specs=[
            pl.BlockSpec((GB, TH, 2, NF), lambda i, j: (i, j, 0, 0)),
            pl.BlockSpec((NAZI, NF), lambda i, j: (0, 0)),
            pl.BlockSpec((NAZI, NF), lambda i, j: (0, 0)),
            pl.BlockSpec((GB, 2, 1, TH), lambda i, j: (i, 0, 0, j)),
            pl.BlockSpec((GB, 1, TH), lambda i, j: (i, 0, j)),
        ],
        out_specs=[
            pl.BlockSpec((NAZI, 1, GB, TH), lambda i, j: (0, i, 0, j)),
            pl.BlockSpec((GB, TH), lambda i, j: (i, j)),
            pl.BlockSpec((1, 1), lambda i, j: (0, 0)),
            pl.BlockSpec((1, 1), lambda i, j: (0, 0)),
        ],
        out_shape=[
            jax.ShapeDtypeStruct((NAZI, NB // GB, GB, NT), jnp.float32),
            jax.ShapeDtypeStruct((NB, NT), jnp.float32),
            jax.ShapeDtypeStruct((1, 1), jnp.float32),
            jax.ShapeDtypeStruct((1, 1), jnp.float32),
        ],
        scratch_shapes=[pltpu.VMEM((1, 128), jnp.float32)],
    )(pred_batch, tmpl_re, tmpl_im, gt3, vad3)

    ss = sst4.reshape(NAZI, NB, NT).transpose(1, 2, 0)
    return ss, doa, acc_a.reshape(()), mae_a.reshape(())


# speedup vs baseline: 4.7714x; 1.0444x over previous
import math

import jax
import jax.numpy as jnp
from jax.experimental import pallas as pl
from jax.experimental.pallas import tpu as pltpu

NB, NT, NF, NAZI = 32, 256, 256, 180
GB = 16
TH = NT // 2
RAD2DEG = 180.0 / math.pi
AZI_STEP = math.pi / (NAZI - 1)


def _fused_kernel(x_ref, wre_ref, wim_ref, gt_ref, vad_ref,
                  sst_ref, doa_ref, acc_a_ref, mae_a_ref, acc_ref):
    i = pl.program_id(0)
    j = pl.program_id(1)
    w_re = wre_ref[...]
    w_im = wim_ref[...]
    azi = (jax.lax.broadcasted_iota(jnp.int32, (NAZI, 1), 0)
           .astype(jnp.float32) * AZI_STEP)
    dims = (((1,), (1,)), ((), ()))
    x_re = x_ref[:, :, 0, :].reshape(GB * TH, NF)
    x_im = x_ref[:, :, 1, :].reshape(GB * TH, NF)
    sst_all = (jax.lax.dot_general(w_re, x_re, dims,
                                   preferred_element_type=jnp.float32)
               + jax.lax.dot_general(w_im, x_im, dims,
                                     preferred_element_type=jnp.float32))
    mx = jnp.max(sst_all, axis=0, keepdims=True)
    doa_all = jnp.min(jnp.where(sst_all >= mx, azi, jnp.inf),
                      axis=0, keepdims=True)
    doa_rows = []
    for bb in range(GB):
        sst_ref[:, 0, bb, :] = sst_all[:, bb * TH:(bb + 1) * TH]
        doa_rows.append(doa_all[:, bb * TH:(bb + 1) * TH])
    doa = jnp.concatenate(doa_rows, axis=0)
    doa_ref[...] = doa

    azi_gt = gt_ref[:, 1, 0, :]
    vad = (vad_ref[:, 0, :] > 0.5).astype(jnp.float32)
    err = jnp.abs(doa - azi_gt) * RAD2DEG
    err = jnp.minimum(err, 360.0 - err)
    corr = (err < 30.0).astype(jnp.float32) * vad
    lane = jax.lax.broadcasted_iota(jnp.int32, (1, 128), 1)
    part = (jnp.where(lane == 0, jnp.sum(corr), 0.0)
            + jnp.where(lane == 1, jnp.sum(vad * err), 0.0)
            + jnp.where(lane == 2, jnp.sum(vad), 0.0))

    first = jnp.logical_and(i == 0, j == 0)
    last = jnp.logical_and(i == NB // GB - 1, j == NT // TH - 1)

    @pl.when(first)
    def _init():
        acc_ref[...] = part

    @pl.when(jnp.logical_not(first))
    def _accum():
        acc_ref[...] += part

    @pl.when(last)
    def _final():
        tot = acc_ref[...]
        denom = jnp.maximum(tot[0, 2], 1.0)
        acc_a_ref[...] = tot[0:1, 0:1] / denom
        mae_a_ref[...] = tot[0:1, 1:2] / denom


def kernel(pred_batch, doa_gt, vad_gt, tmpl_re, tmpl_im, azi_grid):
    del azi_grid
    gt3 = doa_gt.transpose(0, 2, 3, 1)
    vad3 = vad_gt.transpose(0, 2, 1)

    grid = (NB // GB, NT // TH)
    sst4, doa, acc_a, mae_a = pl.pallas_call(
        _fused_kernel,
        grid=grid,
        in_specs=[
            pl.BlockSpec((GB, TH, 2, NF), lambda i, j: (i, j, 0, 0)),
            pl.BlockSpec((NAZI, NF), lambda i, j: (0, 0)),
            pl.BlockSpec((NAZI, NF), lambda i, j: (0, 0)),
            pl.BlockSpec((GB, 2, 1, TH), lambda i, j: (i, 0, 0, j)),
            pl.BlockSpec((GB, 1, TH), lambda i, j: (i, 0, j)),
        ],
        out_specs=[
            pl.BlockSpec((NAZI, 1, GB, TH), lambda i, j: (0, i, 0, j)),
            pl.BlockSpec((GB, TH), lambda i, j: (i, j)),
            pl.BlockSpec((1, 1), lambda i, j: (0, 0)),
            pl.BlockSpec((1, 1), lambda i, j: (0, 0)),
        ],
        out_shape=[
            jax.ShapeDtypeStruct((NAZI, NB // GB, GB, NT), jnp.float32),
            jax.ShapeDtypeStruct((NB, NT), jnp.float32),
            jax.ShapeDtypeStruct((1, 1), jnp.float32),
            jax.ShapeDtypeStruct((1, 1), jnp.float32),
        ],
        scratch_shapes=[pltpu.VMEM((1, 128), jnp.float32)],
    )(pred_batch, tmpl_re, tmpl_im, gt3, vad3)

    ss = sst4.reshape(NAZI, NB, NT).transpose(1, 2, 0)
    return ss, doa, acc_a.reshape(()), mae_a.reshape(())
